# baseline (speedup 1.0000x reference)
import functools

import jax
import jax.numpy as jnp
from jax.experimental import pallas as pl
from jax.experimental.pallas import tpu as pltpu

_ROWS = 2048


def _table_kernel(tok_ref, pos_ref, w_ref, b_ref, lse_ref, posb_ref):
    t = pl.program_id(0)
    posb = jax.lax.dot_general(
        pos_ref[pl.ds(t, 1), :], w_ref[...], (((1,), (0,)), ((), ())),
        preferred_element_type=jnp.float32,
        precision=jax.lax.Precision.DEFAULT) + b_ref[...]
    slab = jax.lax.dot_general(
        tok_ref[...], w_ref[...], (((1,), (0,)), ((), ())),
        preferred_element_type=jnp.float32,
        precision=jax.lax.Precision.DEFAULT) + posb
    m = jnp.max(slab, axis=1, keepdims=True)
    lse = jnp.log(jnp.sum(jnp.exp(slab - m), axis=1, keepdims=True)) + m
    lse_ref[...] = lse[None]
    posb_ref[...] = posb[None]


def _main_kernel(idx_ref, tgt_ref, a1_ref, a2_ref, pos_ref, mask_ref,
                 w_ref, b_ref, logits_ref, part_ref):
    r, v = logits_ref.shape
    c = pos_ref.shape[1]

    ids = idx_ref[...]
    tgt = tgt_ref[...]
    vocab_iota = jax.lax.broadcasted_iota(jnp.int32, (r, v), 1)

    oh1 = (ids == vocab_iota).astype(jnp.bfloat16)
    g1 = jax.lax.dot_general(
        oh1, a1_ref[...], (((1,), (0,)), ((), ())),
        preferred_element_type=jnp.float32,
        precision=jax.lax.Precision.DEFAULT)
    x = g1[:, 0:c] + pos_ref[...]
    lse = jnp.sum(g1[:, c:c + 8] * mask_ref[...], axis=1,
                  keepdims=True)

    oh2 = (tgt == vocab_iota).astype(jnp.bfloat16)
    g2 = jax.lax.dot_general(
        oh2, a2_ref[...], (((1,), (0,)), ((), ())),
        preferred_element_type=jnp.float32,
        precision=jax.lax.Precision.DEFAULT)
    tl = (jnp.sum(x * g2[:, 0:c], axis=1, keepdims=True)
          + jnp.sum(g2[:, c:c + 8] * mask_ref[...], axis=1, keepdims=True))

    logits_ref[...] = jax.lax.dot_general(
        x, w_ref[...], (((1,), (0,)), ((), ())),
        preferred_element_type=jnp.float32,
        precision=jax.lax.Precision.DEFAULT) + b_ref[...]

    part_ref[...] = jnp.full((1, 1, 128), jnp.sum(lse - tl), jnp.float32)


def _loss_reduce_kernel(part_ref, loss_ref, *, n):
    total = jnp.sum(part_ref[...][:, :, 0])
    loss_ref[...] = jnp.full((1, 1), total / n, jnp.float32)


def kernel(idx, targets, tok_table, pos_table, W, b):
    B, T = idx.shape
    V, C = tok_table.shape
    n = B * T
    r = _ROWS
    nblocks = n // r

    idx_r = idx.reshape(n, 1).astype(jnp.int32)
    tgt_r = targets.reshape(n, 1).astype(jnp.int32)
    pos_tile = jnp.tile(pos_table, (r // T, 1))
    mask_tile = jnp.tile(jnp.eye(T, dtype=jnp.float32), (r // T, 1))
    b2 = b.reshape(1, V)

    lse_tab, posb = pl.pallas_call(
        _table_kernel,
        grid=(T,),
        in_specs=[
            pl.BlockSpec((V, C), lambda t: (0, 0)),
            pl.BlockSpec((T, C), lambda t: (0, 0)),
            pl.BlockSpec((C, V), lambda t: (0, 0)),
            pl.BlockSpec((1, V), lambda t: (0, 0)),
        ],
        out_specs=[
            pl.BlockSpec((1, V, 1), lambda t: (t, 0, 0)),
            pl.BlockSpec((1, 1, V), lambda t: (t, 0, 0)),
        ],
        out_shape=[
            jax.ShapeDtypeStruct((T, V, 1), jnp.float32),
            jax.ShapeDtypeStruct((T, 1, V), jnp.float32),
        ],
    )(tok_table, pos_table, W, b2)

    a1 = jnp.concatenate(
        [tok_table, lse_tab.reshape(T, V).T], axis=1).astype(jnp.bfloat16)
    a2 = jnp.concatenate(
        [W.T, posb.reshape(T, V).T], axis=1).astype(jnp.bfloat16)

    logits, parts = pl.pallas_call(
        _main_kernel,
        grid=(nblocks,),
        in_specs=[
            pl.BlockSpec((r, 1), lambda i: (i, 0)),
            pl.BlockSpec((r, 1), lambda i: (i, 0)),
            pl.BlockSpec((V, C + T), lambda i: (0, 0)),
            pl.BlockSpec((V, C + T), lambda i: (0, 0)),
            pl.BlockSpec((r, C), lambda i: (0, 0)),
            pl.BlockSpec((r, T), lambda i: (0, 0)),
            pl.BlockSpec((C, V), lambda i: (0, 0)),
            pl.BlockSpec((1, V), lambda i: (0, 0)),
        ],
        out_specs=[
            pl.BlockSpec((r, V), lambda i: (i, 0)),
            pl.BlockSpec((1, 1, 128), lambda i: (i, 0, 0)),
        ],
        out_shape=[
            jax.ShapeDtypeStruct((n, V), jnp.float32),
            jax.ShapeDtypeStruct((nblocks, 1, 128), jnp.float32),
        ],
        compiler_params=pltpu.CompilerParams(
            dimension_semantics=("parallel",)),
    )(idx_r, tgt_r, a1, a2, pos_tile, mask_tile, W, b2)

    loss = pl.pallas_call(
        functools.partial(_loss_reduce_kernel, n=n),
        out_shape=jax.ShapeDtypeStruct((1, 1), jnp.float32),
    )(parts)
    return logits, loss[0, 0]


# speedup vs baseline: 1.2681x; 1.2681x over previous
import functools

import jax
import jax.numpy as jnp
from jax.experimental import pallas as pl
from jax.experimental.pallas import tpu as pltpu

_ROWS = 2048


def _table_kernel(tok_ref, pos_ref, w_ref, b_ref, lse_ref, posb_ref):
    t = pl.program_id(0)
    posb = jax.lax.dot_general(
        pos_ref[pl.ds(t, 1), :], w_ref[...], (((1,), (0,)), ((), ())),
        preferred_element_type=jnp.float32,
        precision=jax.lax.Precision.DEFAULT) + b_ref[...]
    slab = jax.lax.dot_general(
        tok_ref[...], w_ref[...], (((1,), (0,)), ((), ())),
        preferred_element_type=jnp.float32,
        precision=jax.lax.Precision.DEFAULT) + posb
    m = jnp.max(slab, axis=1, keepdims=True)
    lse = jnp.log(jnp.sum(jnp.exp(slab - m), axis=1, keepdims=True)) + m
    lse_ref[...] = lse[None]
    posb_ref[...] = posb[None]


def _main_kernel(idx_ref, tgt_ref, a1_ref, a2_ref, pos_ref, mask_ref,
                 w_ref, b_ref, logits_ref, part_ref):
    r, v = logits_ref.shape
    c = pos_ref.shape[1]

    ids = idx_ref[...]
    tgt = tgt_ref[...]
    vocab_iota = jax.lax.broadcasted_iota(jnp.int32, (r, v), 1)

    oh1 = (ids == vocab_iota).astype(jnp.bfloat16)
    g1 = jax.lax.dot_general(
        oh1, a1_ref[...], (((1,), (0,)), ((), ())),
        preferred_element_type=jnp.float32,
        precision=jax.lax.Precision.DEFAULT)
    x = g1[:, 0:c] + pos_ref[...]

    oh2 = (tgt == vocab_iota).astype(jnp.bfloat16)
    g2 = jax.lax.dot_general(
        oh2, a2_ref[...], (((1,), (0,)), ((), ())),
        preferred_element_type=jnp.float32,
        precision=jax.lax.Precision.DEFAULT)

    logits_ref[...] = jax.lax.dot_general(
        x, w_ref[...], (((1,), (0,)), ((), ())),
        preferred_element_type=jnp.float32,
        precision=jax.lax.Precision.DEFAULT) + b_ref[...]

    part = (jnp.sum((g1[:, c:c + 8] - g2[:, c:c + 8]) * mask_ref[...])
            - jnp.sum(x * g2[:, 0:c]))
    part_ref[...] = jnp.full((1, 1, 128), part, jnp.float32)


def _loss_reduce_kernel(part_ref, loss_ref, *, n):
    total = jnp.sum(part_ref[...][:, :, 0])
    loss_ref[...] = jnp.full((1, 1), total / n, jnp.float32)


def kernel(idx, targets, tok_table, pos_table, W, b):
    B, T = idx.shape
    V, C = tok_table.shape
    n = B * T
    r = _ROWS
    nblocks = n // r

    idx_r = idx.reshape(n, 1).astype(jnp.int32)
    tgt_r = targets.reshape(n, 1).astype(jnp.int32)
    pos_tile = jnp.tile(pos_table, (r // T, 1))
    mask_tile = jnp.tile(jnp.eye(T, dtype=jnp.float32), (r // T, 1))
    b2 = b.reshape(1, V)

    lse_tab, posb = pl.pallas_call(
        _table_kernel,
        grid=(T,),
        in_specs=[
            pl.BlockSpec((V, C), lambda t: (0, 0)),
            pl.BlockSpec((T, C), lambda t: (0, 0)),
            pl.BlockSpec((C, V), lambda t: (0, 0)),
            pl.BlockSpec((1, V), lambda t: (0, 0)),
        ],
        out_specs=[
            pl.BlockSpec((1, V, 1), lambda t: (t, 0, 0)),
            pl.BlockSpec((1, 1, V), lambda t: (t, 0, 0)),
        ],
        out_shape=[
            jax.ShapeDtypeStruct((T, V, 1), jnp.float32),
            jax.ShapeDtypeStruct((T, 1, V), jnp.float32),
        ],
    )(tok_table, pos_table, W, b2)

    a1 = jnp.concatenate(
        [tok_table, lse_tab.reshape(T, V).T], axis=1).astype(jnp.bfloat16)
    a2 = jnp.concatenate(
        [W.T, posb.reshape(T, V).T], axis=1).astype(jnp.bfloat16)

    logits, parts = pl.pallas_call(
        _main_kernel,
        grid=(nblocks,),
        in_specs=[
            pl.BlockSpec((r, 1), lambda i: (i, 0)),
            pl.BlockSpec((r, 1), lambda i: (i, 0)),
            pl.BlockSpec((V, C + T), lambda i: (0, 0)),
            pl.BlockSpec((V, C + T), lambda i: (0, 0)),
            pl.BlockSpec((r, C), lambda i: (0, 0)),
            pl.BlockSpec((r, T), lambda i: (0, 0)),
            pl.BlockSpec((C, V), lambda i: (0, 0)),
            pl.BlockSpec((1, V), lambda i: (0, 0)),
        ],
        out_specs=[
            pl.BlockSpec((r, V), lambda i: (i, 0)),
            pl.BlockSpec((1, 1, 128), lambda i: (i, 0, 0)),
        ],
        out_shape=[
            jax.ShapeDtypeStruct((n, V), jnp.float32),
            jax.ShapeDtypeStruct((nblocks, 1, 128), jnp.float32),
        ],
        compiler_params=pltpu.CompilerParams(
            dimension_semantics=("parallel",)),
    )(idx_r, tgt_r, a1, a2, pos_tile, mask_tile, W, b2)

    loss = pl.pallas_call(
        functools.partial(_loss_reduce_kernel, n=n),
        out_shape=jax.ShapeDtypeStruct((1, 1), jnp.float32),
    )(parts)
    return logits, loss[0, 0]
